# Initial kernel scaffold; baseline (speedup 1.0000x reference)
import jax
import jax.numpy as jnp
from jax.experimental import pallas as pl


def kernel(x, edge_index, pos_edge_index, neg_edge_index, W_self0, W_neigh0, b0, W_self1, W_neigh1, b1, W_self2, W_neigh2, b2, Wp1, bp1, Wp2, bp2, Wp3, bp3):
    raise NotImplementedError("write your pallas kernel here")


# speedup vs baseline: 2.9849x; 2.9849x over previous
import functools

import jax
import jax.numpy as jnp
from jax import lax
from jax.experimental import pallas as pl
from jax.experimental.pallas import tpu as pltpu
from jax.experimental.pallas import tpu_sc as plsc

N = 10000
NP = 10240
F = 128
E = 320000
PB = 16384
NC = 2
NS = 16
NW = NC * NS

EC = 128
NCHUNK = 80
EPT = NCHUNK * EC
EPAD = EPT * NW
RPT = NP // NS

PC = 128
PCHUNK = PB // NW // PC

_mesh = plsc.VectorSubcoreMesh(core_axis_name="c", subcore_axis_name="s",
                               num_cores=NC, num_subcores=NS)


def _fill(ref, rows, cols, value):
    vec = jnp.full((16,), value, jnp.float32)

    def body(i, _):
        r = i // (cols // 16)
        col = (i % (cols // 16)) * 16
        ref[r, pl.ds(col, 16)] = vec
        return 0

    lax.fori_loop(0, rows * (cols // 16), body, 0)


@functools.partial(
    pl.kernel,
    out_type=jax.ShapeDtypeStruct((NC, NP, F), jnp.float32),
    mesh=_mesh,
    scratch_types=(
        pltpu.VMEM((NCHUNK, EC), jnp.int32),
        pltpu.VMEM((NCHUNK, EC), jnp.int32),
        pltpu.VMEM((EC, F), jnp.float32),
        pltpu.VMEM_SHARED((NP, F), jnp.float32),
        pltpu.SemaphoreType.DMA,
    ),
)
def _sc_aggregate(h_hbm, src_hbm, dst_hbm, out, src_v, dst_v,
                  rows_v, acc_sh, sem):
    c = lax.axis_index("c")
    s = lax.axis_index("s")
    tile = c * NS + s

    pltpu.sync_copy(src_hbm.at[tile], src_v)
    pltpu.sync_copy(dst_hbm.at[tile], dst_v)

    _fill(rows_v, EC, F, 0.0)
    for k in range(RPT // EC):
        pltpu.sync_copy(rows_v, acc_sh.at[pl.ds(s * RPT + k * EC, EC)])
    plsc.subcore_barrier()

    def body(j, _):
        pltpu.async_copy(h_hbm.at[src_v.at[j]], rows_v, sem).wait()
        pltpu.sync_copy(rows_v, acc_sh.at[dst_v.at[j]], add=True)
        return 0

    lax.fori_loop(0, NCHUNK, body, 0)
    plsc.subcore_barrier()

    pltpu.sync_copy(acc_sh.at[pl.ds(s * RPT, RPT)],
                    out.at[c, pl.ds(s * RPT, RPT)])


@functools.partial(
    pl.kernel,
    out_type=jax.ShapeDtypeStruct((NC, NP, F), jnp.float32),
    mesh=_mesh,
    scratch_types=(
        pltpu.VMEM((NCHUNK, EC), jnp.int32),
        pltpu.VMEM((EC, F), jnp.float32),
        pltpu.VMEM_SHARED((NP, F), jnp.float32),
    ),
)
def _sc_degree(dst_hbm, out, dst_v, ones_v, acc_sh):
    c = lax.axis_index("c")
    s = lax.axis_index("s")
    tile = c * NS + s

    pltpu.sync_copy(dst_hbm.at[tile], dst_v)

    _fill(ones_v, EC, F, 0.0)
    for k in range(RPT // EC):
        pltpu.sync_copy(ones_v, acc_sh.at[pl.ds(s * RPT + k * EC, EC)])
    _fill(ones_v, EC, F, 1.0)
    plsc.subcore_barrier()

    def body(j, _):
        pltpu.sync_copy(ones_v, acc_sh.at[dst_v.at[j]], add=True)
        return 0

    lax.fori_loop(0, NCHUNK, body, 0)
    plsc.subcore_barrier()

    pltpu.sync_copy(acc_sh.at[pl.ds(s * RPT, RPT)],
                    out.at[c, pl.ds(s * RPT, RPT)])


@functools.partial(
    pl.kernel,
    out_type=(
        jax.ShapeDtypeStruct((PB, F), jnp.float32),
        jax.ShapeDtypeStruct((PB, F), jnp.float32),
    ),
    mesh=_mesh,
    scratch_types=(
        pltpu.VMEM((PCHUNK, PC), jnp.int32),
        pltpu.VMEM((PCHUNK, PC), jnp.int32),
        pltpu.VMEM((PC, F), jnp.float32),
        pltpu.SemaphoreType.DMA,
    ),
)
def _sc_pair_gather(h_hbm, aidx_hbm, bidx_hbm, outa, outb, a_v, b_v,
                    rows_v, sem):
    c = lax.axis_index("c")
    s = lax.axis_index("s")
    tile = c * NS + s

    pltpu.sync_copy(aidx_hbm.at[tile], a_v)
    pltpu.sync_copy(bidx_hbm.at[tile], b_v)

    def body_a(j, _):
        pltpu.async_copy(h_hbm.at[a_v.at[j]], rows_v, sem).wait()
        pltpu.sync_copy(rows_v, outa.at[pl.ds((tile * PCHUNK + j) * PC, PC)])
        return 0

    lax.fori_loop(0, PCHUNK, body_a, 0)

    def body_b(j, _):
        pltpu.async_copy(h_hbm.at[b_v.at[j]], rows_v, sem).wait()
        pltpu.sync_copy(rows_v, outb.at[pl.ds((tile * PCHUNK + j) * PC, PC)])
        return 0

    lax.fori_loop(0, PCHUNK, body_b, 0)


_LR = 1024


def _tc_layer_body(relu, h_ref, a0_ref, a1_ref, d0_ref, d1_ref, ws_ref,
                   wn_ref, b_ref, o_ref):
    deg = d0_ref[:, :1] + d1_ref[:, :1]
    inv = 1.0 / jnp.maximum(deg, 1.0)
    mean = (a0_ref[...] + a1_ref[...]) * inv
    z = (jnp.dot(h_ref[...], ws_ref[...], preferred_element_type=jnp.float32)
         + jnp.dot(mean, wn_ref[...], preferred_element_type=jnp.float32)
         + b_ref[...])
    o_ref[...] = jnp.maximum(z, 0.0) if relu else z


def _tc_layer(h, a0, a1, d0, d1, ws, wn, b, relu):
    row = lambda i: (i, 0)
    full = lambda i: (0, 0)
    return pl.pallas_call(
        functools.partial(_tc_layer_body, relu),
        grid=(NP // _LR,),
        in_specs=[
            pl.BlockSpec((_LR, F), row),
            pl.BlockSpec((_LR, F), row),
            pl.BlockSpec((_LR, F), row),
            pl.BlockSpec((_LR, F), row),
            pl.BlockSpec((_LR, F), row),
            pl.BlockSpec((F, F), full),
            pl.BlockSpec((F, F), full),
            pl.BlockSpec((1, F), full),
        ],
        out_specs=pl.BlockSpec((_LR, F), row),
        out_shape=jax.ShapeDtypeStruct((NP, F), jnp.float32),
    )(h, a0, a1, d0, d1, ws, wn, b)


_PR = 2048


def _tc_pred_body(a_ref, b_ref, w1_ref, w2_ref, w3_ref, b1_ref, b2_ref,
                  b3_ref, o_ref):
    e = a_ref[...] * b_ref[...]
    e = jnp.maximum(
        jnp.dot(e, w1_ref[...], preferred_element_type=jnp.float32)
        + b1_ref[...], 0.0)
    e = jnp.maximum(
        jnp.dot(e, w2_ref[...], preferred_element_type=jnp.float32)
        + b2_ref[...], 0.0)
    o_ref[...] = (jnp.sum(e * w3_ref[...], axis=1, keepdims=True)
                  + b3_ref[:, :1])


def _tc_pred(a, b, w1, w2, w3row, b1, b2, b3):
    row = lambda i: (i, 0)
    full = lambda i: (0, 0)
    return pl.pallas_call(
        _tc_pred_body,
        grid=(PB // _PR,),
        in_specs=[
            pl.BlockSpec((_PR, F), row),
            pl.BlockSpec((_PR, F), row),
            pl.BlockSpec((F, F), full),
            pl.BlockSpec((F, F), full),
            pl.BlockSpec((1, F), full),
            pl.BlockSpec((1, F), full),
            pl.BlockSpec((1, F), full),
            pl.BlockSpec((1, F), full),
        ],
        out_specs=pl.BlockSpec((_PR, 1), row),
        out_shape=jax.ShapeDtypeStruct((PB, 1), jnp.float32),
    )(a, b, w1, w2, w3row, b1, b2, b3)


def kernel(x, edge_index, pos_edge_index, neg_edge_index,
           W_self0, W_neigh0, b0, W_self1, W_neigh1, b1,
           W_self2, W_neigh2, b2, Wp1, bp1, Wp2, bp2, Wp3, bp3):
    f32 = jnp.float32
    i32 = jnp.int32
    npad = EPAD - E
    src = jnp.concatenate(
        [edge_index[0].astype(i32), jnp.zeros((npad,), i32)]
    ).reshape(NW, NCHUNK, EC)
    dst = jnp.concatenate(
        [edge_index[1].astype(i32), jnp.full((npad,), N, i32)]
    ).reshape(NW, NCHUNK, EC)

    degs = _sc_degree(dst)
    d0, d1 = degs[0], degs[1]

    h = jnp.zeros((NP, F), f32).at[:N].set(x.astype(f32))
    layers = [
        (W_self0, W_neigh0, b0, True),
        (W_self1, W_neigh1, b1, True),
        (W_self2, W_neigh2, b2, False),
    ]
    for ws, wn, bb, relu in layers:
        aggs = _sc_aggregate(h, src, dst)
        a0, a1 = aggs[0], aggs[1]
        h = _tc_layer(h, a0, a1, d0, d1, ws, wn, bb.reshape(1, F), relu)

    aidx = jnp.concatenate([pos_edge_index[0], neg_edge_index[0]])
    bidx = jnp.concatenate([pos_edge_index[1], neg_edge_index[1]])
    aidx = aidx.astype(i32).reshape(NW, PCHUNK, PC)
    bidx = bidx.astype(i32).reshape(NW, PCHUNK, PC)

    ha, hb = _sc_pair_gather(h, aidx, bidx)

    out = _tc_pred(ha, hb, Wp1, Wp2, Wp3.reshape(1, F),
                   bp1.reshape(1, F), bp2.reshape(1, F),
                   jnp.broadcast_to(bp3.reshape(1, 1), (1, F)))
    return out[:PB // 2], out[PB // 2:]


# speedup vs baseline: 3.0212x; 1.0122x over previous
import functools

import jax
import jax.numpy as jnp
from jax import lax
from jax.experimental import pallas as pl
from jax.experimental.pallas import tpu as pltpu
from jax.experimental.pallas import tpu_sc as plsc

N = 10000
NP = 10240
F = 128
E = 320000
PB = 16384
NC = 2
NS = 16
NW = NC * NS

EC = 128
NCHUNK = 80
EPT = NCHUNK * EC
EPAD = EPT * NW
RPT = NP // NS

PC = 128
PCHUNK = PB // NW // PC

_mesh = plsc.VectorSubcoreMesh(core_axis_name="c", subcore_axis_name="s",
                               num_cores=NC, num_subcores=NS)


def _fill(ref, rows, cols, value):
    vec = jnp.full((16,), value, jnp.float32)

    def body(i, _):
        r = i // (cols // 16)
        col = (i % (cols // 16)) * 16
        ref[r, pl.ds(col, 16)] = vec
        return 0

    lax.fori_loop(0, rows * (cols // 16), body, 0)


NBUF = 2
BN = 16
NBLK = NCHUNK // BN


@functools.partial(
    pl.kernel,
    out_type=jax.ShapeDtypeStruct((NC, NP, F), jnp.float32),
    mesh=_mesh,
    scratch_types=(
        pltpu.VMEM((BN, EC), jnp.int32),
        pltpu.VMEM((BN, EC), jnp.int32),
        pltpu.VMEM((EC, F), jnp.float32),
        pltpu.VMEM((EC, F), jnp.float32),
        pltpu.VMEM_SHARED((NP, F), jnp.float32),
        pltpu.SemaphoreType.DMA,
        pltpu.SemaphoreType.DMA,
        pltpu.SemaphoreType.DMA,
    ),
)
def _sc_aggregate(h_hbm, src_hbm, dst_hbm, out, src_v, dst_v,
                  b0, b1, acc_sh, g0, g1, ssem):
    bufs = (b0, b1)
    gsems = (g0, g1)
    c = lax.axis_index("c")
    s = lax.axis_index("s")
    tile = c * NS + s

    _fill(b0, EC, F, 0.0)
    for k in range(RPT // EC):
        pltpu.sync_copy(b0, acc_sh.at[pl.ds(s * RPT + k * EC, EC)])
    plsc.subcore_barrier()

    for blk in range(NBLK):
        pltpu.sync_copy(src_hbm.at[tile, pl.ds(blk * BN, BN)], src_v)
        pltpu.sync_copy(dst_hbm.at[tile, pl.ds(blk * BN, BN)], dst_v)

        def body(k, _):
            j = k * NBUF
            gs = [pltpu.async_copy(h_hbm.at[src_v.at[j + b]], bufs[b],
                                   gsems[b]) for b in range(NBUF)]
            ss = []
            for b in range(NBUF):
                gs[b].wait()
                ss.append(pltpu.async_copy(bufs[b],
                                           acc_sh.at[dst_v.at[j + b]],
                                           ssem, add=True))
            for sc in ss:
                sc.wait()
            return 0

        lax.fori_loop(0, BN // NBUF, body, 0)
    plsc.subcore_barrier()

    pltpu.sync_copy(acc_sh.at[pl.ds(s * RPT, RPT)],
                    out.at[c, pl.ds(s * RPT, RPT)])


@functools.partial(
    pl.kernel,
    out_type=jax.ShapeDtypeStruct((NC, NP, F), jnp.float32),
    mesh=_mesh,
    scratch_types=(
        pltpu.VMEM((NCHUNK, EC), jnp.int32),
        pltpu.VMEM((EC, F), jnp.float32),
        pltpu.VMEM_SHARED((NP, F), jnp.float32),
    ),
)
def _sc_degree(dst_hbm, out, dst_v, ones_v, acc_sh):
    c = lax.axis_index("c")
    s = lax.axis_index("s")
    tile = c * NS + s

    pltpu.sync_copy(dst_hbm.at[tile], dst_v)

    _fill(ones_v, EC, F, 0.0)
    for k in range(RPT // EC):
        pltpu.sync_copy(ones_v, acc_sh.at[pl.ds(s * RPT + k * EC, EC)])
    _fill(ones_v, EC, F, 1.0)
    plsc.subcore_barrier()

    def body(j, _):
        pltpu.sync_copy(ones_v, acc_sh.at[dst_v.at[j]], add=True)
        return 0

    lax.fori_loop(0, NCHUNK, body, 0)
    plsc.subcore_barrier()

    pltpu.sync_copy(acc_sh.at[pl.ds(s * RPT, RPT)],
                    out.at[c, pl.ds(s * RPT, RPT)])


@functools.partial(
    pl.kernel,
    out_type=(
        jax.ShapeDtypeStruct((PB, F), jnp.float32),
        jax.ShapeDtypeStruct((PB, F), jnp.float32),
    ),
    mesh=_mesh,
    scratch_types=(
        pltpu.VMEM((PCHUNK, PC), jnp.int32),
        pltpu.VMEM((PCHUNK, PC), jnp.int32),
        pltpu.VMEM((PC, F), jnp.float32),
        pltpu.SemaphoreType.DMA,
    ),
)
def _sc_pair_gather(h_hbm, aidx_hbm, bidx_hbm, outa, outb, a_v, b_v,
                    rows_v, sem):
    c = lax.axis_index("c")
    s = lax.axis_index("s")
    tile = c * NS + s

    pltpu.sync_copy(aidx_hbm.at[tile], a_v)
    pltpu.sync_copy(bidx_hbm.at[tile], b_v)

    def body_a(j, _):
        pltpu.async_copy(h_hbm.at[a_v.at[j]], rows_v, sem).wait()
        pltpu.sync_copy(rows_v, outa.at[pl.ds((tile * PCHUNK + j) * PC, PC)])
        return 0

    lax.fori_loop(0, PCHUNK, body_a, 0)

    def body_b(j, _):
        pltpu.async_copy(h_hbm.at[b_v.at[j]], rows_v, sem).wait()
        pltpu.sync_copy(rows_v, outb.at[pl.ds((tile * PCHUNK + j) * PC, PC)])
        return 0

    lax.fori_loop(0, PCHUNK, body_b, 0)


_LR = 1024


def _tc_layer_body(relu, h_ref, a0_ref, a1_ref, d0_ref, d1_ref, ws_ref,
                   wn_ref, b_ref, o_ref):
    deg = d0_ref[:, :1] + d1_ref[:, :1]
    inv = 1.0 / jnp.maximum(deg, 1.0)
    mean = (a0_ref[...] + a1_ref[...]) * inv
    z = (jnp.dot(h_ref[...], ws_ref[...], preferred_element_type=jnp.float32)
         + jnp.dot(mean, wn_ref[...], preferred_element_type=jnp.float32)
         + b_ref[...])
    o_ref[...] = jnp.maximum(z, 0.0) if relu else z


def _tc_layer(h, a0, a1, d0, d1, ws, wn, b, relu):
    row = lambda i: (i, 0)
    full = lambda i: (0, 0)
    return pl.pallas_call(
        functools.partial(_tc_layer_body, relu),
        grid=(NP // _LR,),
        in_specs=[
            pl.BlockSpec((_LR, F), row),
            pl.BlockSpec((_LR, F), row),
            pl.BlockSpec((_LR, F), row),
            pl.BlockSpec((_LR, F), row),
            pl.BlockSpec((_LR, F), row),
            pl.BlockSpec((F, F), full),
            pl.BlockSpec((F, F), full),
            pl.BlockSpec((1, F), full),
        ],
        out_specs=pl.BlockSpec((_LR, F), row),
        out_shape=jax.ShapeDtypeStruct((NP, F), jnp.float32),
    )(h, a0, a1, d0, d1, ws, wn, b)


_PR = 2048


def _tc_pred_body(a_ref, b_ref, w1_ref, w2_ref, w3_ref, b1_ref, b2_ref,
                  b3_ref, o_ref):
    e = a_ref[...] * b_ref[...]
    e = jnp.maximum(
        jnp.dot(e, w1_ref[...], preferred_element_type=jnp.float32)
        + b1_ref[...], 0.0)
    e = jnp.maximum(
        jnp.dot(e, w2_ref[...], preferred_element_type=jnp.float32)
        + b2_ref[...], 0.0)
    o_ref[...] = (jnp.sum(e * w3_ref[...], axis=1, keepdims=True)
                  + b3_ref[:, :1])


def _tc_pred(a, b, w1, w2, w3row, b1, b2, b3):
    row = lambda i: (i, 0)
    full = lambda i: (0, 0)
    return pl.pallas_call(
        _tc_pred_body,
        grid=(PB // _PR,),
        in_specs=[
            pl.BlockSpec((_PR, F), row),
            pl.BlockSpec((_PR, F), row),
            pl.BlockSpec((F, F), full),
            pl.BlockSpec((F, F), full),
            pl.BlockSpec((1, F), full),
            pl.BlockSpec((1, F), full),
            pl.BlockSpec((1, F), full),
            pl.BlockSpec((1, F), full),
        ],
        out_specs=pl.BlockSpec((_PR, 1), row),
        out_shape=jax.ShapeDtypeStruct((PB, 1), jnp.float32),
    )(a, b, w1, w2, w3row, b1, b2, b3)


def kernel(x, edge_index, pos_edge_index, neg_edge_index,
           W_self0, W_neigh0, b0, W_self1, W_neigh1, b1,
           W_self2, W_neigh2, b2, Wp1, bp1, Wp2, bp2, Wp3, bp3):
    f32 = jnp.float32
    i32 = jnp.int32
    npad = EPAD - E
    src = jnp.concatenate(
        [edge_index[0].astype(i32), jnp.zeros((npad,), i32)]
    ).reshape(NW, NCHUNK, EC)
    dst = jnp.concatenate(
        [edge_index[1].astype(i32), jnp.full((npad,), N, i32)]
    ).reshape(NW, NCHUNK, EC)

    degs = _sc_degree(dst)
    d0, d1 = degs[0], degs[1]

    h = jnp.zeros((NP, F), f32).at[:N].set(x.astype(f32))
    layers = [
        (W_self0, W_neigh0, b0, True),
        (W_self1, W_neigh1, b1, True),
        (W_self2, W_neigh2, b2, False),
    ]
    for ws, wn, bb, relu in layers:
        aggs = _sc_aggregate(h, src, dst)
        a0, a1 = aggs[0], aggs[1]
        h = _tc_layer(h, a0, a1, d0, d1, ws, wn, bb.reshape(1, F), relu)

    aidx = jnp.concatenate([pos_edge_index[0], neg_edge_index[0]])
    bidx = jnp.concatenate([pos_edge_index[1], neg_edge_index[1]])
    aidx = aidx.astype(i32).reshape(NW, PCHUNK, PC)
    bidx = bidx.astype(i32).reshape(NW, PCHUNK, PC)

    ha, hb = _sc_pair_gather(h, aidx, bidx)

    out = _tc_pred(ha, hb, Wp1, Wp2, Wp3.reshape(1, F),
                   bp1.reshape(1, F), bp2.reshape(1, F),
                   jnp.broadcast_to(bp3.reshape(1, 1), (1, F)))
    return out[:PB // 2], out[PB // 2:]


# speedup vs baseline: 7.8338x; 2.5929x over previous
import functools

import jax
import jax.numpy as jnp
from jax import lax
from jax.experimental import pallas as pl
from jax.experimental.pallas import tpu as pltpu
from jax.experimental.pallas import tpu_sc as plsc

N = 10000
NP = 10240
F = 128
E = 320000
PB = 16384
NC = 2
NS = 16
NW = NC * NS

EC = 128
NCHUNK = 80
EPT = NCHUNK * EC
EPAD = EPT * NW
RPT = NP // NS

PC = 128
PCHUNK = PB // NW // PC

_mesh = plsc.VectorSubcoreMesh(core_axis_name="c", subcore_axis_name="s",
                               num_cores=NC, num_subcores=NS)


def _fill(ref, rows, cols, value):
    vec = jnp.full((16,), value, jnp.float32)

    def body(i, _):
        r = i // (cols // 16)
        col = (i % (cols // 16)) * 16
        ref[r, pl.ds(col, 16)] = vec
        return 0

    lax.fori_loop(0, rows * (cols // 16), body, 0)


NBUF = 2
BN = 16
NBLK = NCHUNK // BN


@functools.partial(
    pl.kernel,
    out_type=jax.ShapeDtypeStruct((NC, NP, F), jnp.float32),
    mesh=_mesh,
    scratch_types=(
        pltpu.VMEM((BN, EC), jnp.int32),
        pltpu.VMEM((BN, EC), jnp.int32),
        pltpu.VMEM((EC, F), jnp.float32),
        pltpu.VMEM((EC, F), jnp.float32),
        pltpu.VMEM_SHARED((NP, F), jnp.float32),
        pltpu.SemaphoreType.DMA,
        pltpu.SemaphoreType.DMA,
        pltpu.SemaphoreType.DMA,
    ),
)
def _sc_aggregate(h_hbm, src_hbm, dst_hbm, out, src_v, dst_v,
                  b0, b1, acc_sh, g0, g1, ssem):
    bufs = (b0, b1)
    gsems = (g0, g1)
    c = lax.axis_index("c")
    s = lax.axis_index("s")
    tile = c * NS + s

    _fill(b0, EC, F, 0.0)
    for k in range(RPT // EC):
        pltpu.sync_copy(b0, acc_sh.at[pl.ds(s * RPT + k * EC, EC)])
    plsc.subcore_barrier()

    for blk in range(NBLK):
        pltpu.sync_copy(src_hbm.at[tile, pl.ds(blk * BN, BN)], src_v)
        pltpu.sync_copy(dst_hbm.at[tile, pl.ds(blk * BN, BN)], dst_v)

        def body(k, _):
            j = k * NBUF
            gs = [pltpu.async_copy(h_hbm.at[src_v.at[j + b]], bufs[b],
                                   gsems[b]) for b in range(NBUF)]
            ss = []
            for b in range(NBUF):
                gs[b].wait()
                ss.append(pltpu.async_copy(bufs[b],
                                           acc_sh.at[dst_v.at[j + b]],
                                           ssem, add=True))
            for sc in ss:
                sc.wait()
            return 0

        lax.fori_loop(0, BN // NBUF, body, 0)
    plsc.subcore_barrier()

    pltpu.sync_copy(acc_sh.at[pl.ds(s * RPT, RPT)],
                    out.at[c, pl.ds(s * RPT, RPT)])


@functools.partial(
    pl.kernel,
    out_type=jax.ShapeDtypeStruct((NC, NP, F), jnp.float32),
    mesh=_mesh,
    scratch_types=(
        pltpu.VMEM((NCHUNK, EC), jnp.int32),
        pltpu.VMEM((EC, F), jnp.float32),
        pltpu.VMEM_SHARED((NP, F), jnp.float32),
    ),
)
def _sc_degree(dst_hbm, out, dst_v, ones_v, acc_sh):
    c = lax.axis_index("c")
    s = lax.axis_index("s")
    tile = c * NS + s

    pltpu.sync_copy(dst_hbm.at[tile], dst_v)

    _fill(ones_v, EC, F, 0.0)
    for k in range(RPT // EC):
        pltpu.sync_copy(ones_v, acc_sh.at[pl.ds(s * RPT + k * EC, EC)])
    _fill(ones_v, EC, F, 1.0)
    plsc.subcore_barrier()

    def body(j, _):
        pltpu.sync_copy(ones_v, acc_sh.at[dst_v.at[j]], add=True)
        return 0

    lax.fori_loop(0, NCHUNK, body, 0)
    plsc.subcore_barrier()

    pltpu.sync_copy(acc_sh.at[pl.ds(s * RPT, RPT)],
                    out.at[c, pl.ds(s * RPT, RPT)])


@functools.partial(
    pl.kernel,
    out_type=(
        jax.ShapeDtypeStruct((PB, F), jnp.float32),
        jax.ShapeDtypeStruct((PB, F), jnp.float32),
    ),
    mesh=_mesh,
    scratch_types=(
        pltpu.VMEM((PCHUNK, PC), jnp.int32),
        pltpu.VMEM((PCHUNK, PC), jnp.int32),
        pltpu.VMEM((PC, F), jnp.float32),
        pltpu.SemaphoreType.DMA,
    ),
)
def _sc_pair_gather(h_hbm, aidx_hbm, bidx_hbm, outa, outb, a_v, b_v,
                    rows_v, sem):
    c = lax.axis_index("c")
    s = lax.axis_index("s")
    tile = c * NS + s

    pltpu.sync_copy(aidx_hbm.at[tile], a_v)
    pltpu.sync_copy(bidx_hbm.at[tile], b_v)

    def body_a(j, _):
        pltpu.async_copy(h_hbm.at[a_v.at[j]], rows_v, sem).wait()
        pltpu.sync_copy(rows_v, outa.at[pl.ds((tile * PCHUNK + j) * PC, PC)])
        return 0

    lax.fori_loop(0, PCHUNK, body_a, 0)

    def body_b(j, _):
        pltpu.async_copy(h_hbm.at[b_v.at[j]], rows_v, sem).wait()
        pltpu.sync_copy(rows_v, outb.at[pl.ds((tile * PCHUNK + j) * PC, PC)])
        return 0

    lax.fori_loop(0, PCHUNK, body_b, 0)


_LR = 1024


def _tc_layer_body(relu, h_ref, a0_ref, a1_ref, d0_ref, d1_ref, ws_ref,
                   wn_ref, b_ref, o_ref):
    deg = d0_ref[:, :1] + d1_ref[:, :1]
    inv = 1.0 / jnp.maximum(deg, 1.0)
    mean = (a0_ref[...] + a1_ref[...]) * inv
    z = (jnp.dot(h_ref[...], ws_ref[...], preferred_element_type=jnp.float32)
         + jnp.dot(mean, wn_ref[...], preferred_element_type=jnp.float32)
         + b_ref[...])
    o_ref[...] = jnp.maximum(z, 0.0) if relu else z


def _tc_layer(h, a0, a1, d0, d1, ws, wn, b, relu):
    row = lambda i: (i, 0)
    full = lambda i: (0, 0)
    return pl.pallas_call(
        functools.partial(_tc_layer_body, relu),
        grid=(NP // _LR,),
        in_specs=[
            pl.BlockSpec((_LR, F), row),
            pl.BlockSpec((_LR, F), row),
            pl.BlockSpec((_LR, F), row),
            pl.BlockSpec((_LR, F), row),
            pl.BlockSpec((_LR, F), row),
            pl.BlockSpec((F, F), full),
            pl.BlockSpec((F, F), full),
            pl.BlockSpec((1, F), full),
        ],
        out_specs=pl.BlockSpec((_LR, F), row),
        out_shape=jax.ShapeDtypeStruct((NP, F), jnp.float32),
    )(h, a0, a1, d0, d1, ws, wn, b)


_PR = 2048


def _tc_pred_body(a_ref, b_ref, w1_ref, w2_ref, w3_ref, b1_ref, b2_ref,
                  b3_ref, o_ref):
    e = a_ref[...] * b_ref[...]
    e = jnp.maximum(
        jnp.dot(e, w1_ref[...], preferred_element_type=jnp.float32)
        + b1_ref[...], 0.0)
    e = jnp.maximum(
        jnp.dot(e, w2_ref[...], preferred_element_type=jnp.float32)
        + b2_ref[...], 0.0)
    o_ref[...] = (jnp.sum(e * w3_ref[...], axis=1, keepdims=True)
                  + b3_ref[:, :1])


def _tc_pred(a, b, w1, w2, w3row, b1, b2, b3):
    row = lambda i: (i, 0)
    full = lambda i: (0, 0)
    return pl.pallas_call(
        _tc_pred_body,
        grid=(PB // _PR,),
        in_specs=[
            pl.BlockSpec((_PR, F), row),
            pl.BlockSpec((_PR, F), row),
            pl.BlockSpec((F, F), full),
            pl.BlockSpec((F, F), full),
            pl.BlockSpec((1, F), full),
            pl.BlockSpec((1, F), full),
            pl.BlockSpec((1, F), full),
            pl.BlockSpec((1, F), full),
        ],
        out_specs=pl.BlockSpec((_PR, 1), row),
        out_shape=jax.ShapeDtypeStruct((PB, 1), jnp.float32),
    )(a, b, w1, w2, w3row, b1, b2, b3)


def kernel(x, edge_index, pos_edge_index, neg_edge_index,
           W_self0, W_neigh0, b0, W_self1, W_neigh1, b1,
           W_self2, W_neigh2, b2, Wp1, bp1, Wp2, bp2, Wp3, bp3):
    f32 = jnp.float32
    i32 = jnp.int32
    npad = EPAD - E
    pad_iota = lax.iota(i32, npad)
    src = jnp.concatenate(
        [edge_index[0].astype(i32), pad_iota % N]
    ).reshape(NW, NCHUNK, EC)
    dst = jnp.concatenate(
        [edge_index[1].astype(i32), N + pad_iota % (NP - N)]
    ).reshape(NW, NCHUNK, EC)

    degs = _sc_degree(dst)
    d0, d1 = degs[0], degs[1]

    h = jnp.zeros((NP, F), f32).at[:N].set(x.astype(f32))
    layers = [
        (W_self0, W_neigh0, b0, True),
        (W_self1, W_neigh1, b1, True),
        (W_self2, W_neigh2, b2, False),
    ]
    for ws, wn, bb, relu in layers:
        aggs = _sc_aggregate(h, src, dst)
        a0, a1 = aggs[0], aggs[1]
        h = _tc_layer(h, a0, a1, d0, d1, ws, wn, bb.reshape(1, F), relu)

    aidx = jnp.concatenate([pos_edge_index[0], neg_edge_index[0]])
    bidx = jnp.concatenate([pos_edge_index[1], neg_edge_index[1]])
    aidx = aidx.astype(i32).reshape(NW, PCHUNK, PC)
    bidx = bidx.astype(i32).reshape(NW, PCHUNK, PC)

    ha, hb = _sc_pair_gather(h, aidx, bidx)

    out = _tc_pred(ha, hb, Wp1, Wp2, Wp3.reshape(1, F),
                   bp1.reshape(1, F), bp2.reshape(1, F),
                   jnp.broadcast_to(bp3.reshape(1, 1), (1, F)))
    return out[:PB // 2], out[PB // 2:]


# speedup vs baseline: 8.1415x; 1.0393x over previous
import functools

import jax
import jax.numpy as jnp
from jax import lax
from jax.experimental import pallas as pl
from jax.experimental.pallas import tpu as pltpu
from jax.experimental.pallas import tpu_sc as plsc

N = 10000
NP = 10240
F = 128
E = 320000
PB = 16384
NC = 2
NS = 16
NW = NC * NS

EC = 128
NCHUNK = 80
EPT = NCHUNK * EC
EPAD = EPT * NW
RPT = NP // NS

PC = 128
PCHUNK = PB // NW // PC

_mesh = plsc.VectorSubcoreMesh(core_axis_name="c", subcore_axis_name="s",
                               num_cores=NC, num_subcores=NS)


def _fill(ref, rows, cols, value):
    vec = jnp.full((16,), value, jnp.float32)

    def body(i, _):
        r = i // (cols // 16)
        col = (i % (cols // 16)) * 16
        ref[r, pl.ds(col, 16)] = vec
        return 0

    lax.fori_loop(0, rows * (cols // 16), body, 0)


NBUF = 2
BN = 16
NBLK = NCHUNK // BN


@functools.partial(
    pl.kernel,
    out_type=jax.ShapeDtypeStruct((NC, NP, F), jnp.float32),
    mesh=_mesh,
    scratch_types=(
        pltpu.VMEM((BN, EC), jnp.int32),
        pltpu.VMEM((BN, EC), jnp.int32),
        pltpu.VMEM((EC, F), jnp.float32),
        pltpu.VMEM((EC, F), jnp.float32),
        pltpu.VMEM_SHARED((NP, F), jnp.float32),
        pltpu.SemaphoreType.DMA,
        pltpu.SemaphoreType.DMA,
        pltpu.SemaphoreType.DMA,
    ),
)
def _sc_aggregate(h_hbm, src_hbm, dst_hbm, out, src_v, dst_v,
                  b0, b1, acc_sh, g0, g1, ssem):
    bufs = (b0, b1)
    gsems = (g0, g1)
    c = lax.axis_index("c")
    s = lax.axis_index("s")
    tile = c * NS + s

    _fill(b0, EC, F, 0.0)
    for k in range(RPT // EC):
        pltpu.sync_copy(b0, acc_sh.at[pl.ds(s * RPT + k * EC, EC)])
    plsc.subcore_barrier()

    for blk in range(NBLK):
        pltpu.sync_copy(src_hbm.at[tile, pl.ds(blk * BN, BN)], src_v)
        pltpu.sync_copy(dst_hbm.at[tile, pl.ds(blk * BN, BN)], dst_v)

        def body(k, _):
            j = k * NBUF
            gs = [pltpu.async_copy(h_hbm.at[src_v.at[j + b]], bufs[b],
                                   gsems[b]) for b in range(NBUF)]
            ss = []
            for b in range(NBUF):
                gs[b].wait()
                ss.append(pltpu.async_copy(bufs[b],
                                           acc_sh.at[dst_v.at[j + b]],
                                           ssem, add=True))
            for sc in ss:
                sc.wait()
            return 0

        lax.fori_loop(0, BN // NBUF, body, 0)
    plsc.subcore_barrier()

    pltpu.sync_copy(acc_sh.at[pl.ds(s * RPT, RPT)],
                    out.at[c, pl.ds(s * RPT, RPT)])


@functools.partial(
    pl.kernel,
    out_type=jax.ShapeDtypeStruct((NC, NP, F), jnp.float32),
    mesh=_mesh,
    scratch_types=(
        pltpu.VMEM((NCHUNK, EC), jnp.int32),
        pltpu.VMEM((EC, F), jnp.float32),
        pltpu.VMEM_SHARED((NP, F), jnp.float32),
    ),
)
def _sc_degree(dst_hbm, out, dst_v, ones_v, acc_sh):
    c = lax.axis_index("c")
    s = lax.axis_index("s")
    tile = c * NS + s

    pltpu.sync_copy(dst_hbm.at[tile], dst_v)

    _fill(ones_v, EC, F, 0.0)
    for k in range(RPT // EC):
        pltpu.sync_copy(ones_v, acc_sh.at[pl.ds(s * RPT + k * EC, EC)])
    _fill(ones_v, EC, F, 1.0)
    plsc.subcore_barrier()

    def body(j, _):
        pltpu.sync_copy(ones_v, acc_sh.at[dst_v.at[j]], add=True)
        return 0

    lax.fori_loop(0, NCHUNK, body, 0)
    plsc.subcore_barrier()

    pltpu.sync_copy(acc_sh.at[pl.ds(s * RPT, RPT)],
                    out.at[c, pl.ds(s * RPT, RPT)])


@functools.partial(
    pl.kernel,
    out_type=(
        jax.ShapeDtypeStruct((PB, F), jnp.float32),
        jax.ShapeDtypeStruct((PB, F), jnp.float32),
    ),
    mesh=_mesh,
    scratch_types=(
        pltpu.VMEM((PCHUNK, PC), jnp.int32),
        pltpu.VMEM((PCHUNK, PC), jnp.int32),
        pltpu.VMEM((PC, F), jnp.float32),
        pltpu.SemaphoreType.DMA,
    ),
)
def _sc_pair_gather(h_hbm, aidx_hbm, bidx_hbm, outa, outb, a_v, b_v,
                    rows_v, sem):
    c = lax.axis_index("c")
    s = lax.axis_index("s")
    tile = c * NS + s

    pltpu.sync_copy(aidx_hbm.at[tile], a_v)
    pltpu.sync_copy(bidx_hbm.at[tile], b_v)

    def body_a(j, _):
        pltpu.async_copy(h_hbm.at[a_v.at[j]], rows_v, sem).wait()
        pltpu.sync_copy(rows_v, outa.at[pl.ds((tile * PCHUNK + j) * PC, PC)])
        return 0

    lax.fori_loop(0, PCHUNK, body_a, 0)

    def body_b(j, _):
        pltpu.async_copy(h_hbm.at[b_v.at[j]], rows_v, sem).wait()
        pltpu.sync_copy(rows_v, outb.at[pl.ds((tile * PCHUNK + j) * PC, PC)])
        return 0

    lax.fori_loop(0, PCHUNK, body_b, 0)


_LR = 1024


def _tc_layer_body(relu, h_ref, a_ref, d_ref, ws_ref, wn_ref, b_ref, o_ref):
    deg = d_ref[0, :, :1] + d_ref[1, :, :1]
    inv = 1.0 / jnp.maximum(deg, 1.0)
    mean = (a_ref[0] + a_ref[1]) * inv
    z = (jnp.dot(h_ref[...], ws_ref[...], preferred_element_type=jnp.float32)
         + jnp.dot(mean, wn_ref[...], preferred_element_type=jnp.float32)
         + b_ref[...])
    o_ref[...] = jnp.maximum(z, 0.0) if relu else z


def _tc_layer(h, aggs, degs, ws, wn, b, relu):
    row = lambda i: (i, 0)
    prow = lambda i: (0, i, 0)
    full = lambda i: (0, 0)
    return pl.pallas_call(
        functools.partial(_tc_layer_body, relu),
        grid=(NP // _LR,),
        in_specs=[
            pl.BlockSpec((_LR, F), row),
            pl.BlockSpec((NC, _LR, F), prow),
            pl.BlockSpec((NC, _LR, F), prow),
            pl.BlockSpec((F, F), full),
            pl.BlockSpec((F, F), full),
            pl.BlockSpec((1, F), full),
        ],
        out_specs=pl.BlockSpec((_LR, F), row),
        out_shape=jax.ShapeDtypeStruct((NP, F), jnp.float32),
    )(h, aggs, degs, ws, wn, b)


_PR = 2048


def _tc_pred_body(a_ref, b_ref, w1_ref, w2_ref, w3_ref, b1_ref, b2_ref,
                  b3_ref, o_ref):
    e = a_ref[...] * b_ref[...]
    e = jnp.maximum(
        jnp.dot(e, w1_ref[...], preferred_element_type=jnp.float32)
        + b1_ref[...], 0.0)
    e = jnp.maximum(
        jnp.dot(e, w2_ref[...], preferred_element_type=jnp.float32)
        + b2_ref[...], 0.0)
    o_ref[...] = (jnp.sum(e * w3_ref[...], axis=1, keepdims=True)
                  + b3_ref[:, :1])


def _tc_pred(a, b, w1, w2, w3row, b1, b2, b3):
    row = lambda i: (i, 0)
    full = lambda i: (0, 0)
    return pl.pallas_call(
        _tc_pred_body,
        grid=(PB // _PR,),
        in_specs=[
            pl.BlockSpec((_PR, F), row),
            pl.BlockSpec((_PR, F), row),
            pl.BlockSpec((F, F), full),
            pl.BlockSpec((F, F), full),
            pl.BlockSpec((1, F), full),
            pl.BlockSpec((1, F), full),
            pl.BlockSpec((1, F), full),
            pl.BlockSpec((1, F), full),
        ],
        out_specs=pl.BlockSpec((_PR, 1), row),
        out_shape=jax.ShapeDtypeStruct((PB, 1), jnp.float32),
    )(a, b, w1, w2, w3row, b1, b2, b3)


def kernel(x, edge_index, pos_edge_index, neg_edge_index,
           W_self0, W_neigh0, b0, W_self1, W_neigh1, b1,
           W_self2, W_neigh2, b2, Wp1, bp1, Wp2, bp2, Wp3, bp3):
    f32 = jnp.float32
    i32 = jnp.int32
    npad = EPAD - E
    pad_iota = lax.iota(i32, npad)
    src = jnp.concatenate(
        [edge_index[0].astype(i32), pad_iota % N]
    ).reshape(NW, NCHUNK, EC)
    dst = jnp.concatenate(
        [edge_index[1].astype(i32), N + pad_iota % (NP - N)]
    ).reshape(NW, NCHUNK, EC)

    degs = _sc_degree(dst)

    h = jnp.zeros((NP, F), f32).at[:N].set(x.astype(f32))
    layers = [
        (W_self0, W_neigh0, b0, True),
        (W_self1, W_neigh1, b1, True),
        (W_self2, W_neigh2, b2, False),
    ]
    for ws, wn, bb, relu in layers:
        aggs = _sc_aggregate(h, src, dst)
        h = _tc_layer(h, aggs, degs, ws, wn, bb.reshape(1, F), relu)

    aidx = jnp.concatenate([pos_edge_index[0], neg_edge_index[0]])
    bidx = jnp.concatenate([pos_edge_index[1], neg_edge_index[1]])
    aidx = aidx.astype(i32).reshape(NW, PCHUNK, PC)
    bidx = bidx.astype(i32).reshape(NW, PCHUNK, PC)

    ha, hb = _sc_pair_gather(h, aidx, bidx)

    out = _tc_pred(ha, hb, Wp1, Wp2, Wp3.reshape(1, F),
                   bp1.reshape(1, F), bp2.reshape(1, F),
                   jnp.broadcast_to(bp3.reshape(1, 1), (1, F)))
    return out[:PB // 2], out[PB // 2:]
